# Initial kernel scaffold; baseline (speedup 1.0000x reference)
import functools

import jax
import jax.numpy as jnp
from jax import lax
from jax.experimental import pallas as pl
from jax.experimental.pallas import tpu as pltpu
from jax.experimental.pallas import tpu_sc as plsc

N_NODES = 10000
N_PAD = 10016
E_EDGES = 320000
D = 128
NUM_CORES = 2
NUM_SUBCORES = 16
NW = NUM_CORES * NUM_SUBCORES
CHUNK = 128
EPW = -(-E_EDGES // (NW * CHUNK)) * CHUNK
E_PAD = EPW * NW
NCHUNKS = EPW // CHUNK
ROWS_PER_TILE = N_PAD // NUM_SUBCORES

_mesh = plsc.VectorSubcoreMesh(core_axis_name="c", subcore_axis_name="s")


@functools.partial(
    pl.kernel,
    out_type=jax.ShapeDtypeStruct((NW, 2 * N_PAD), jnp.float32),
    mesh=_mesh,
    scratch_types=[
        pltpu.VMEM((EPW,), jnp.int32),
        pltpu.VMEM((EPW,), jnp.int32),
        pltpu.VMEM((2 * N_PAD,), jnp.float32),
    ],
)
def _degree_kernel(src_hbm, dst_hbm, degp_hbm, src_v, dst_v, deg_v):
    cid = lax.axis_index("c")
    sid = lax.axis_index("s")
    wid = sid * NUM_CORES + cid
    base = wid * EPW
    pltpu.sync_copy(src_hbm.at[pl.ds(base, EPW)], src_v)
    pltpu.sync_copy(dst_hbm.at[pl.ds(base, EPW)], dst_v)

    zeros16 = jnp.zeros((16,), jnp.float32)

    def zero_body(i, c):
        deg_v[pl.ds(i * 16, 16)] = zeros16
        return c

    lax.fori_loop(0, (2 * N_PAD) // 16, zero_body, 0)

    ones16 = jnp.ones((16,), jnp.float32)
    off16 = jnp.full((16,), N_PAD, jnp.int32)

    def body(i, c):
        s = src_v[pl.ds(i * 16, 16)]
        d = dst_v[pl.ds(i * 16, 16)]
        plsc.addupdate_scatter(deg_v, [s], ones16)
        plsc.addupdate_scatter(deg_v, [d + off16], ones16)
        return c

    lax.fori_loop(0, EPW // 16, body, 0)
    pltpu.sync_copy(deg_v, degp_hbm.at[wid])


@functools.partial(
    pl.kernel,
    out_type=jax.ShapeDtypeStruct((NUM_CORES, N_PAD, D), jnp.float32),
    mesh=_mesh,
    scratch_types=[
        pltpu.VMEM((CHUNK,), jnp.int32),
        pltpu.VMEM((CHUNK,), jnp.int32),
        pltpu.VMEM((CHUNK, D), jnp.float32),
        pltpu.VMEM_SHARED((N_PAD, D), jnp.float32),
        pltpu.SemaphoreType.DMA,
    ],
)
def _agg_kernel(xw_hbm, src_hbm, dst_hbm, zero_hbm, out_hbm,
                sidx, didx, rows, acc_sh, sem):
    cid = lax.axis_index("c")
    sid = lax.axis_index("s")
    wid = sid * NUM_CORES + cid
    base = wid * EPW
    rows_lo = sid * ROWS_PER_TILE

    pltpu.sync_copy(zero_hbm.at[pl.ds(rows_lo, ROWS_PER_TILE)],
                    acc_sh.at[pl.ds(rows_lo, ROWS_PER_TILE)])
    plsc.subcore_barrier()

    def body(c, carry):
        off = base + c * CHUNK
        pltpu.sync_copy(src_hbm.at[pl.ds(off, CHUNK)], sidx)
        pltpu.async_copy(xw_hbm.at[sidx], rows, sem).wait()
        pltpu.sync_copy(dst_hbm.at[pl.ds(off, CHUNK)], didx)
        pltpu.sync_copy(rows, acc_sh.at[didx], add=True)
        return carry

    lax.fori_loop(0, NCHUNKS, body, 0)
    plsc.subcore_barrier()

    pltpu.sync_copy(acc_sh.at[pl.ds(rows_lo, ROWS_PER_TILE)],
                    out_hbm.at[cid, pl.ds(rows_lo, ROWS_PER_TILE)])


def _norm_matmul_body(degpt_ref, feats_ref, w1_ref, xw_ref, norm_ref):
    deg = jnp.sum(degpt_ref[...], axis=1, keepdims=True)
    norm = lax.rsqrt(jnp.maximum(deg, 1.0))
    norm_ref[...] = norm
    ns = norm[:N_PAD]
    xw_ref[...] = jnp.dot(feats_ref[...] * ns, w1_ref[...],
                          preferred_element_type=jnp.float32)


def _mid_layer_body(aggp_ref, norm_ref, b1_ref, w2_ref, h1_ref, xw2_ref):
    agg = aggp_ref[0] + aggp_ref[1]
    norm = norm_ref[...]
    ns = norm[:N_PAD]
    nd = norm[N_PAD:]
    h1 = jnp.maximum(agg * nd + b1_ref[...], 0.0)
    h1_ref[...] = h1
    xw2_ref[...] = jnp.dot(h1 * ns, w2_ref[...],
                           preferred_element_type=jnp.float32)


def _final_body(aggp_ref, norm_ref, b2_ref, h2_ref):
    agg = aggp_ref[0] + aggp_ref[1]
    nd = norm_ref[...][N_PAD:]
    h2_ref[...] = agg * nd + b2_ref[...]


_norm_matmul = pl.pallas_call(
    _norm_matmul_body,
    out_shape=(jax.ShapeDtypeStruct((N_PAD, D), jnp.float32),
               jax.ShapeDtypeStruct((2 * N_PAD, 1), jnp.float32)),
)

_mid_layer = pl.pallas_call(
    _mid_layer_body,
    out_shape=(jax.ShapeDtypeStruct((N_PAD, D), jnp.float32),
               jax.ShapeDtypeStruct((N_PAD, D), jnp.float32)),
)

_final_layer = pl.pallas_call(
    _final_body,
    out_shape=jax.ShapeDtypeStruct((N_PAD, D), jnp.float32),
)


def kernel(feats, edge_index, W1, b1, W2, b2):
    src = edge_index[0].astype(jnp.int32)
    dst = edge_index[1].astype(jnp.int32)
    pad = E_PAD - E_EDGES
    dummy = jnp.full((pad,), N_NODES, jnp.int32)
    srcp = jnp.concatenate([src, dummy])
    dstp = jnp.concatenate([dst, dummy])
    featsp = jnp.concatenate(
        [feats, jnp.zeros((N_PAD - N_NODES, D), jnp.float32)], axis=0)
    zero_rows = jnp.zeros((N_PAD, D), jnp.float32)

    degp = _degree_kernel(srcp, dstp)
    degpt = degp.T
    xw1, norm = _norm_matmul(degpt, featsp, W1)
    agg1p = _agg_kernel(xw1, srcp, dstp, zero_rows)
    h1, xw2 = _mid_layer(agg1p, norm, b1.reshape(1, D), W2)
    agg2p = _agg_kernel(xw2, srcp, dstp, zero_rows)
    h2 = _final_layer(agg2p, norm, b2.reshape(1, D))

    h1o = h1[:N_NODES]
    h2o = h2[:N_NODES]
    return ((h1o, h2o), h2o)


# speedup vs baseline: 3.7125x; 3.7125x over previous
import functools

import jax
import jax.numpy as jnp
from jax import lax
from jax.experimental import pallas as pl
from jax.experimental.pallas import tpu as pltpu
from jax.experimental.pallas import tpu_sc as plsc

N_NODES = 10000
N_PAD = 10112
E_EDGES = 320000
D = 128
NUM_CORES = 2
NUM_SUBCORES = 16
NW = NUM_CORES * NUM_SUBCORES
CHUNK = 128
EPW = -(-E_EDGES // (NW * CHUNK)) * CHUNK
E_PAD = EPW * NW
NCHUNKS = EPW // CHUNK
ROWS_PER_TILE = N_PAD // NUM_SUBCORES
NP_DEG = 10112

@functools.cache
def _get_degree_kernel():
    mesh = plsc.VectorSubcoreMesh(
        core_axis_name="c", subcore_axis_name="s",
        num_cores=NUM_CORES, num_subcores=NUM_SUBCORES)
    return functools.partial(
        pl.kernel,
        out_type=jax.ShapeDtypeStruct((NUM_CORES, N_PAD, D), jnp.float32),
        mesh=mesh,
        scratch_types=[
            pltpu.VMEM((CHUNK,), jnp.int32),
            pltpu.VMEM((CHUNK,), jnp.int32),
            pltpu.VMEM((CHUNK, D), jnp.float32),
            pltpu.VMEM((CHUNK, D), jnp.float32),
            pltpu.VMEM_SHARED((N_PAD, D), jnp.float32),
        ],
    )(_degree_kernel_body)


def _degree_kernel_body(src_hbm, dst_hbm, ones_s_hbm, ones_d_hbm, zeros_hbm,
                        degp_hbm, sidx, didx, ones_s, ones_d, acc):
    cid = lax.axis_index("c")
    sid = lax.axis_index("s")
    wid = sid * NUM_CORES + cid
    base = wid * EPW
    rows_lo = sid * ROWS_PER_TILE

    pltpu.sync_copy(zeros_hbm.at[pl.ds(rows_lo, ROWS_PER_TILE)],
                    acc.at[pl.ds(rows_lo, ROWS_PER_TILE)])
    pltpu.sync_copy(ones_s_hbm, ones_s)
    pltpu.sync_copy(ones_d_hbm, ones_d)
    plsc.subcore_barrier()

    def body(c, carry):
        off = base + c * CHUNK
        pltpu.sync_copy(src_hbm.at[pl.ds(off, CHUNK)], sidx)
        pltpu.sync_copy(ones_s, acc.at[sidx], add=True)
        pltpu.sync_copy(dst_hbm.at[pl.ds(off, CHUNK)], didx)
        pltpu.sync_copy(ones_d, acc.at[didx], add=True)
        return carry

    lax.fori_loop(0, NCHUNKS, body, 0)
    plsc.subcore_barrier()

    pltpu.sync_copy(acc.at[pl.ds(rows_lo, ROWS_PER_TILE)],
                    degp_hbm.at[cid, pl.ds(rows_lo, ROWS_PER_TILE)])


@functools.cache
def _get_agg_kernel():
    mesh = plsc.VectorSubcoreMesh(
        core_axis_name="c", subcore_axis_name="s",
        num_cores=NUM_CORES, num_subcores=NUM_SUBCORES)
    return functools.partial(
        pl.kernel,
        out_type=jax.ShapeDtypeStruct((NUM_CORES, N_PAD, D), jnp.float32),
        mesh=mesh,
        scratch_types=[
            pltpu.VMEM((CHUNK,), jnp.int32),
            pltpu.VMEM((CHUNK,), jnp.int32),
            pltpu.VMEM((CHUNK, D), jnp.float32),
            pltpu.VMEM_SHARED((N_PAD, D), jnp.float32),
            pltpu.SemaphoreType.DMA,
        ],
    )(_agg_kernel_body)


def _agg_kernel_body(xw_hbm, src_hbm, dst_hbm, zero_hbm, out_hbm,
                     sidx, didx, rows, acc_sh, sem):
    cid = lax.axis_index("c")
    sid = lax.axis_index("s")
    wid = sid * NUM_CORES + cid
    base = wid * EPW
    rows_lo = sid * ROWS_PER_TILE

    pltpu.sync_copy(zero_hbm.at[pl.ds(rows_lo, ROWS_PER_TILE)],
                    acc_sh.at[pl.ds(rows_lo, ROWS_PER_TILE)])
    plsc.subcore_barrier()

    def body(c, carry):
        off = base + c * CHUNK
        pltpu.sync_copy(src_hbm.at[pl.ds(off, CHUNK)], sidx)
        pltpu.async_copy(xw_hbm.at[sidx], rows, sem).wait()
        pltpu.sync_copy(dst_hbm.at[pl.ds(off, CHUNK)], didx)
        pltpu.sync_copy(rows, acc_sh.at[didx], add=True)
        return carry

    lax.fori_loop(0, NCHUNKS, body, 0)
    plsc.subcore_barrier()

    pltpu.sync_copy(acc_sh.at[pl.ds(rows_lo, ROWS_PER_TILE)],
                    out_hbm.at[cid, pl.ds(rows_lo, ROWS_PER_TILE)])


def _norm_matmul_body(degp_ref, feats_ref, w1_ref, xw_ref, norm_ref):
    deg_d = degp_ref[0, :, 0:1] + degp_ref[1, :, 0:1]
    deg_s = degp_ref[0, :, 64:65] + degp_ref[1, :, 64:65]
    ns = lax.rsqrt(jnp.maximum(deg_s, 1.0))
    nd = lax.rsqrt(jnp.maximum(deg_d, 1.0))
    norm_ref[...] = jnp.concatenate([ns, nd], axis=1)
    xw_ref[...] = jnp.dot(feats_ref[...] * ns, w1_ref[...],
                          preferred_element_type=jnp.float32)


def _mid_layer_body(aggp_ref, norm_ref, b1_ref, w2_ref, h1_ref, xw2_ref):
    agg = aggp_ref[0] + aggp_ref[1]
    norm = norm_ref[...]
    ns = norm[:, 0:1]
    nd = norm[:, 1:2]
    h1 = jnp.maximum(agg * nd + b1_ref[...], 0.0)
    h1_ref[...] = h1
    xw2_ref[...] = jnp.dot(h1 * ns, w2_ref[...],
                           preferred_element_type=jnp.float32)


def _final_body(aggp_ref, norm_ref, b2_ref, h2_ref):
    agg = aggp_ref[0] + aggp_ref[1]
    nd = norm_ref[...][:, 1:2]
    h2_ref[...] = agg * nd + b2_ref[...]


_norm_matmul = pl.pallas_call(
    _norm_matmul_body,
    out_shape=(jax.ShapeDtypeStruct((N_PAD, D), jnp.float32),
               jax.ShapeDtypeStruct((N_PAD, 2), jnp.float32)),
)

_mid_layer = pl.pallas_call(
    _mid_layer_body,
    out_shape=(jax.ShapeDtypeStruct((N_PAD, D), jnp.float32),
               jax.ShapeDtypeStruct((N_PAD, D), jnp.float32)),
)

_final_layer = pl.pallas_call(
    _final_body,
    out_shape=jax.ShapeDtypeStruct((N_PAD, D), jnp.float32),
)


def kernel(feats, edge_index, W1, b1, W2, b2):
    src = edge_index[0].astype(jnp.int32)
    dst = edge_index[1].astype(jnp.int32)
    pad = E_PAD - E_EDGES
    dummy = jnp.full((pad,), N_NODES, jnp.int32)
    srcp = jnp.concatenate([src, dummy])
    dstp = jnp.concatenate([dst, dummy])
    featsp = jnp.concatenate(
        [feats, jnp.zeros((N_PAD - N_NODES, D), jnp.float32)], axis=0)
    zero_rows = jnp.zeros((N_PAD, D), jnp.float32)
    lane = jnp.arange(D, dtype=jnp.int32)[None, :]
    ones_s = jnp.broadcast_to((lane >= 64).astype(jnp.float32), (CHUNK, D))
    ones_d = jnp.broadcast_to((lane < 64).astype(jnp.float32), (CHUNK, D))

    degree_kernel = _get_degree_kernel()
    agg_kernel = _get_agg_kernel()
    degp = degree_kernel(srcp, dstp, ones_s, ones_d, zero_rows)
    xw1, norm = _norm_matmul(degp, featsp, W1)
    agg1p = agg_kernel(xw1, srcp, dstp, zero_rows)
    h1, xw2 = _mid_layer(agg1p, norm, b1.reshape(1, D), W2)
    agg2p = agg_kernel(xw2, srcp, dstp, zero_rows)
    h2 = _final_layer(agg2p, norm, b2.reshape(1, D))

    h1o = h1[:N_NODES]
    h2o = h2[:N_NODES]
    return ((h1o, h2o), h2o)
